# baseline (speedup 1.0000x reference)
import functools

import jax
import jax.numpy as jnp
from jax.experimental import pallas as pl
from jax.experimental.pallas import tpu as pltpu

_DB = 2048
_K = 10


def _retrieve_kernel(n_docs, q_ref, d_ref, idx_ref, score_ref,
                     qn_ref, s_ref, run_s, run_i, mprev, aprev):
    blk = pl.program_id(0)
    nblk = pl.num_programs(0)
    nq = q_ref.shape[0]

    @pl.when(blk == 0)
    def _init():
        q = q_ref[...]
        qn = jnp.sqrt(jnp.sum(q * q, axis=-1, keepdims=True))
        qn_ref[...] = q / jnp.maximum(qn, 1e-12)
        run_s[...] = jnp.full(run_s.shape, -jnp.inf, dtype=run_s.dtype)
        run_i[...] = jnp.zeros(run_i.shape, dtype=run_i.dtype)

    d = d_ref[...]
    dn = jnp.sqrt(jnp.sum(d * d, axis=-1, keepdims=True))
    d = d / jnp.maximum(dn, 1e-12)

    s = jax.lax.dot_general(qn_ref[...], d, (((1,), (1,)), ((), ())),
                            preferred_element_type=jnp.float32)

    lane = jax.lax.broadcasted_iota(jnp.int32, s.shape, 1)
    valid = (blk * _DB + lane) < n_docs
    s = jnp.where(valid, s, -jnp.inf)

    ch = 128
    nch = _DB // ch
    lane128 = jax.lax.broadcasted_iota(jnp.int32, (nq, ch), 1)
    imax = jnp.full((nq, ch), jnp.iinfo(jnp.int32).max, dtype=jnp.int32)

    def _max_argmax(sv):
        m_ch = sv[:, 0:ch]
        g_ch = lane128
        for k in range(1, nch):
            sk = sv[:, k * ch:(k + 1) * ch]
            cgt = sk > m_ch
            m_ch = jnp.where(cgt, sk, m_ch)
            g_ch = jnp.where(cgt, lane128 + k * ch, g_ch)
        m = jnp.max(m_ch, axis=1, keepdims=True)
        am = jnp.min(jnp.where(m_ch == m, g_ch, imax), axis=1, keepdims=True)
        return m, am

    def _insert(m, am):
        gi = am + blk * _DB
        rs = run_s[...]
        ri = run_i[...]
        prev_s = jnp.concatenate(
            [jnp.full((nq, 1), jnp.inf, dtype=rs.dtype), rs[:, :-1]], axis=1)
        prev_i = jnp.concatenate(
            [jnp.zeros((nq, 1), dtype=ri.dtype), ri[:, :-1]], axis=1)
        keep = rs >= m
        ins = prev_s >= m
        run_s[...] = jnp.where(keep, rs, jnp.where(ins, m, prev_s))
        run_i[...] = jnp.where(keep, ri, jnp.where(ins, gi, prev_i))

    def _more(m):
        t = run_s[:, _K - 1:_K]
        return jnp.max(jnp.where(m > t, jnp.int32(1), jnp.int32(0))) > 0

    m, am = _max_argmax(s)
    _insert(m, am)
    for _ in range(1, 3):
        s = jnp.where(lane == am, -jnp.inf, s)
        m, am = _max_argmax(s)
        _insert(m, am)

    mprev[...] = m
    aprev[...] = am

    @pl.when(_more(m))
    def _round4():
        sd = jnp.where(lane == am, -jnp.inf, s)
        s_ref[...] = sd
        md, amd = _max_argmax(sd)
        _insert(md, amd)
        mprev[...] = md
        aprev[...] = amd

    for _ in range(4, _K):
        @pl.when(_more(mprev[...]))
        def _roundk():
            sd = jnp.where(lane == aprev[...], -jnp.inf, s_ref[...])
            s_ref[...] = sd
            md, amd = _max_argmax(sd)
            _insert(md, amd)
            mprev[...] = md
            aprev[...] = amd

    @pl.when(blk == nblk - 1)
    def _emit():
        idx_ref[...] = run_i[...]
        score_ref[...] = run_s[...]


def kernel(query_embeds, doc_embeds, top_k):
    del top_k
    nq, dim = query_embeds.shape
    n_docs = doc_embeds.shape[0]
    nblk = pl.cdiv(n_docs, _DB)
    idx, scores = pl.pallas_call(
        functools.partial(_retrieve_kernel, n_docs),
        grid=(nblk,),
        in_specs=[
            pl.BlockSpec((nq, dim), lambda i: (0, 0)),
            pl.BlockSpec((_DB, dim), lambda i: (i, 0)),
        ],
        out_specs=[
            pl.BlockSpec((nq, _K), lambda i: (0, 0)),
            pl.BlockSpec((nq, _K), lambda i: (0, 0)),
        ],
        out_shape=[
            jax.ShapeDtypeStruct((nq, _K), jnp.int32),
            jax.ShapeDtypeStruct((nq, _K), jnp.float32),
        ],
        scratch_shapes=[
            pltpu.VMEM((nq, dim), jnp.float32),
            pltpu.VMEM((nq, _DB), jnp.float32),
            pltpu.VMEM((nq, _K), jnp.float32),
            pltpu.VMEM((nq, _K), jnp.int32),
            pltpu.VMEM((nq, 1), jnp.float32),
            pltpu.VMEM((nq, 1), jnp.int32),
        ],
        compiler_params=pltpu.CompilerParams(
            dimension_semantics=("arbitrary",)),
    )(query_embeds, doc_embeds)
    return idx, scores


# speedup vs baseline: 1.2296x; 1.2296x over previous
import functools

import jax
import jax.numpy as jnp
from jax.experimental import pallas as pl
from jax.experimental.pallas import tpu as pltpu

_DB = 2048
_K = 10


def _retrieve_kernel(n_docs, q_ref, d_ref, idx_ref, score_ref,
                     qn_ref, s_ref, run_s, run_i, aprev):
    blk = pl.program_id(0)
    nblk = pl.num_programs(0)
    nq = q_ref.shape[0]

    @pl.when(blk == 0)
    def _init():
        q = q_ref[...]
        qn = jnp.sqrt(jnp.sum(q * q, axis=-1, keepdims=True))
        qn_ref[...] = q / jnp.maximum(qn, 1e-12)
        run_s[...] = jnp.full(run_s.shape, -jnp.inf, dtype=run_s.dtype)
        run_i[...] = jnp.zeros(run_i.shape, dtype=run_i.dtype)

    d = d_ref[...]
    dn = jnp.sqrt(jnp.sum(d * d, axis=-1, keepdims=True))
    d = d / jnp.maximum(dn, 1e-12)

    s = jax.lax.dot_general(qn_ref[...], d, (((1,), (1,)), ((), ())),
                            preferred_element_type=jnp.float32)

    lane = jax.lax.broadcasted_iota(jnp.int32, s.shape, 1)
    valid = (blk * _DB + lane) < n_docs
    s = jnp.where(valid, s, -jnp.inf)

    ch = 128
    nch = _DB // ch
    lane128 = jax.lax.broadcasted_iota(jnp.int32, (nq, ch), 1)
    imax = jnp.full((nq, ch), jnp.iinfo(jnp.int32).max, dtype=jnp.int32)

    def _max_argmax(sv, thresh=None):
        m_ch = sv[:, 0:ch]
        g_ch = lane128
        acc = None
        if thresh is not None:
            acc = (m_ch > thresh).astype(jnp.int32)
        for k in range(1, nch):
            sk = sv[:, k * ch:(k + 1) * ch]
            cgt = sk > m_ch
            m_ch = jnp.where(cgt, sk, m_ch)
            g_ch = jnp.where(cgt, lane128 + k * ch, g_ch)
            if acc is not None:
                acc = acc + (sk > thresh).astype(jnp.int32)
        m = jnp.max(m_ch, axis=1, keepdims=True)
        am = jnp.min(jnp.where(m_ch == m, g_ch, imax), axis=1, keepdims=True)
        if acc is None:
            return m, am
        return m, am, jnp.max(jnp.sum(acc, axis=1))

    def _insert(m, am):
        gi = am + blk * _DB
        rs = run_s[...]
        ri = run_i[...]
        prev_s = jnp.concatenate(
            [jnp.full((nq, 1), jnp.inf, dtype=rs.dtype), rs[:, :-1]], axis=1)
        prev_i = jnp.concatenate(
            [jnp.zeros((nq, 1), dtype=ri.dtype), ri[:, :-1]], axis=1)
        keep = rs >= m
        ins = prev_s >= m
        run_s[...] = jnp.where(keep, rs, jnp.where(ins, m, prev_s))
        run_i[...] = jnp.where(keep, ri, jnp.where(ins, gi, prev_i))

    m, am, c = _max_argmax(s, thresh=run_s[:, _K - 1:_K])
    _insert(m, am)
    for _ in range(1, 3):
        s = jnp.where(lane == am, -jnp.inf, s)
        m, am = _max_argmax(s)
        _insert(m, am)

    aprev[...] = am

    @pl.when(c >= 4)
    def _round4():
        sd = jnp.where(lane == am, -jnp.inf, s)
        s_ref[...] = sd
        md, amd = _max_argmax(sd)
        _insert(md, amd)
        aprev[...] = amd

    for k in range(5, _K + 1):
        @pl.when(c >= k)
        def _roundk(k=k):
            sd = jnp.where(lane == aprev[...], -jnp.inf, s_ref[...])
            md, amd = _max_argmax(sd)
            _insert(md, amd)
            if k < _K:
                s_ref[...] = sd
                aprev[...] = amd

    @pl.when(blk == nblk - 1)
    def _emit():
        idx_ref[...] = run_i[...]
        score_ref[...] = run_s[...]


def kernel(query_embeds, doc_embeds, top_k):
    del top_k
    nq, dim = query_embeds.shape
    n_docs = doc_embeds.shape[0]
    nblk = pl.cdiv(n_docs, _DB)
    idx, scores = pl.pallas_call(
        functools.partial(_retrieve_kernel, n_docs),
        grid=(nblk,),
        in_specs=[
            pl.BlockSpec((nq, dim), lambda i: (0, 0)),
            pl.BlockSpec((_DB, dim), lambda i: (i, 0)),
        ],
        out_specs=[
            pl.BlockSpec((nq, _K), lambda i: (0, 0)),
            pl.BlockSpec((nq, _K), lambda i: (0, 0)),
        ],
        out_shape=[
            jax.ShapeDtypeStruct((nq, _K), jnp.int32),
            jax.ShapeDtypeStruct((nq, _K), jnp.float32),
        ],
        scratch_shapes=[
            pltpu.VMEM((nq, dim), jnp.float32),
            pltpu.VMEM((nq, _DB), jnp.float32),
            pltpu.VMEM((nq, _K), jnp.float32),
            pltpu.VMEM((nq, _K), jnp.int32),
            pltpu.VMEM((nq, 1), jnp.int32),
        ],
        compiler_params=pltpu.CompilerParams(
            dimension_semantics=("arbitrary",)),
    )(query_embeds, doc_embeds)
    return idx, scores
